# Initial kernel scaffold; baseline (speedup 1.0000x reference)
import jax
import jax.numpy as jnp
from jax.experimental import pallas as pl


def kernel(x, emb_weight):
    raise NotImplementedError("write your pallas kernel here")


# speedup vs baseline: 1.3701x; 1.3701x over previous
import functools
import math

import jax
import jax.numpy as jnp
from jax import lax
from jax.experimental import pallas as pl
from jax.experimental.pallas import tpu as pltpu
from jax.experimental.pallas import tpu_sc as plsc


def _swish_kernel(w_ref, o_ref):
    w = w_ref[...]
    o_ref[...] = w * jax.nn.sigmoid(w)


def _activate_table(w):
    return pl.pallas_call(
        _swish_kernel,
        out_shape=jax.ShapeDtypeStruct(w.shape, w.dtype),
    )(w)


CHUNK = 80


@functools.lru_cache(maxsize=None)
def _make_gather(n, d):
    info = plsc.get_sparse_core_info()
    nc, ns = info.num_cores, info.num_subcores
    nw = nc * ns
    assert n % CHUNK == 0
    n_chunks = n // CHUNK
    max_per_tile = -(-n_chunks // nw)
    mesh = plsc.VectorSubcoreMesh(core_axis_name="c", subcore_axis_name="s")

    @functools.partial(
        pl.kernel,
        out_type=jax.ShapeDtypeStruct((n, d), jnp.float32),
        mesh=mesh,
        scratch_types=[
            pltpu.VMEM((CHUNK,), jnp.int32),
            pltpu.VMEM((CHUNK, d), jnp.float32),
            pltpu.SemaphoreType.DMA,
        ],
    )
    def gather_kernel(table_hbm, idx_hbm, out_hbm, idx_v, rows_v, sem):
        wid = lax.axis_index("s") * nc + lax.axis_index("c")
        for t in range(max_per_tile):
            j = wid + t * nw

            @pl.when(j < n_chunks)
            def _():
                base = pl.multiple_of(j * CHUNK, CHUNK)
                pltpu.sync_copy(idx_hbm.at[pl.ds(base, CHUNK)], idx_v)
                pltpu.async_copy(table_hbm.at[idx_v], rows_v, sem).wait()
                pltpu.sync_copy(rows_v, out_hbm.at[pl.ds(base, CHUNK)])

    return gather_kernel


def kernel(x, emb_weight):
    act = _activate_table(emb_weight)
    idx = x.astype(jnp.int32)
    return _make_gather(idx.shape[0], emb_weight.shape[1])(act, idx)


# speedup vs baseline: 1.4446x; 1.0544x over previous
import functools

import jax
import jax.numpy as jnp
from jax import lax
from jax.experimental import pallas as pl
from jax.experimental.pallas import tpu as pltpu
from jax.experimental.pallas import tpu_sc as plsc


def _swish_kernel(w_ref, o_ref):
    w = w_ref[...]
    o_ref[...] = w * jax.nn.sigmoid(w)


def _activate_table(w):
    return pl.pallas_call(
        _swish_kernel,
        out_shape=jax.ShapeDtypeStruct(w.shape, w.dtype),
    )(w)


CHUNK = 80
NBUF = 4


@functools.lru_cache(maxsize=None)
def _make_gather(n, d):
    info = plsc.get_sparse_core_info()
    nc, ns = info.num_cores, info.num_subcores
    nw = nc * ns
    assert n % CHUNK == 0
    n_chunks = n // CHUNK
    trips = -(-n_chunks // nw)
    mesh = plsc.VectorSubcoreMesh(core_axis_name="c", subcore_axis_name="s")

    @functools.partial(
        pl.kernel,
        out_type=jax.ShapeDtypeStruct((n, d), jnp.float32),
        mesh=mesh,
        scratch_types=[
            pltpu.VMEM((NBUF, CHUNK), jnp.int32),
            pltpu.VMEM((NBUF, CHUNK, d), jnp.float32),
            [pltpu.SemaphoreType.DMA] * NBUF,
            [pltpu.SemaphoreType.DMA] * NBUF,
            [pltpu.SemaphoreType.DMA] * NBUF,
        ],
    )
    def gather_kernel(table_hbm, idx_hbm, out_hbm, idx_v, rows_v,
                      isems, gsems, osems):
        wid = lax.axis_index("s") * nc + lax.axis_index("c")

        def base(t):
            j = wid + t * nw
            if (t + 1) * nw > n_chunks:
                j = jnp.where(j < n_chunks, j, wid)
            return pl.multiple_of(j * CHUNK, CHUNK)

        def start_i(t):
            b = t % NBUF
            return pltpu.async_copy(
                idx_hbm.at[pl.ds(base(t), CHUNK)], idx_v.at[b], isems[b])

        def start_g(t):
            b = t % NBUF
            return pltpu.async_copy(
                table_hbm.at[idx_v.at[b]], rows_v.at[b], gsems[b])

        def start_o(t):
            b = t % NBUF
            return pltpu.async_copy(
                rows_v.at[b], out_hbm.at[pl.ds(base(t), CHUNK)], osems[b])

        icopies = [start_i(t) for t in range(min(NBUF, trips))]
        gcopies = [None] * trips
        ocopies = [None] * trips
        for t in range(trips):
            if t >= NBUF:
                ocopies[t - NBUF].wait()
            icopies[t].wait()
            gcopies[t] = start_g(t)
            if t >= 1:
                gcopies[t - 1].wait()
                ocopies[t - 1] = start_o(t - 1)
                if t - 1 + NBUF < trips:
                    icopies.append(start_i(t - 1 + NBUF))
        gcopies[trips - 1].wait()
        ocopies[trips - 1] = start_o(trips - 1)
        for t in range(max(0, trips - NBUF + 1), trips):
            ocopies[t].wait()

    return gather_kernel


def kernel(x, emb_weight):
    act = _activate_table(emb_weight)
    idx = x.astype(jnp.int32)
    return _make_gather(idx.shape[0], emb_weight.shape[1])(act, idx)


# speedup vs baseline: 1.4715x; 1.0186x over previous
import functools

import jax
import jax.numpy as jnp
from jax import lax
from jax.experimental import pallas as pl
from jax.experimental.pallas import tpu as pltpu
from jax.experimental.pallas import tpu_sc as plsc


def _swish_kernel(w_ref, o_ref):
    w = w_ref[...]
    o_ref[...] = w * jax.nn.sigmoid(w)


def _activate_table(w):
    return pl.pallas_call(
        _swish_kernel,
        out_shape=jax.ShapeDtypeStruct(w.shape, w.dtype),
    )(w)


CHUNK = 128
NBUF = 4


@functools.lru_cache(maxsize=None)
def _make_gather(n, d):
    info = plsc.get_sparse_core_info()
    nc, ns = info.num_cores, info.num_subcores
    nw = nc * ns
    assert n % 8 == 0 and n >= CHUNK
    n_chunks = -(-n // CHUNK)
    trips = -(-n_chunks // nw)
    mesh = plsc.VectorSubcoreMesh(core_axis_name="c", subcore_axis_name="s")

    @functools.partial(
        pl.kernel,
        out_type=jax.ShapeDtypeStruct((n, d), jnp.float32),
        mesh=mesh,
        scratch_types=[
            pltpu.VMEM((NBUF, CHUNK), jnp.int32),
            pltpu.VMEM((NBUF, CHUNK, d), jnp.float32),
            [pltpu.SemaphoreType.DMA] * NBUF,
            [pltpu.SemaphoreType.DMA] * NBUF,
            [pltpu.SemaphoreType.DMA] * NBUF,
        ],
    )
    def gather_kernel(table_hbm, idx_hbm, out_hbm, idx_v, rows_v,
                      isems, gsems, osems):
        wid = lax.axis_index("s") * nc + lax.axis_index("c")

        def base(t):
            j = wid + t * nw
            if (t + 1) * nw > n_chunks:
                j = jnp.where(j < n_chunks, j, wid)
            b = j * CHUNK
            if n % CHUNK != 0:
                b = jnp.minimum(b, n - CHUNK)
            return pl.multiple_of(b, 8)

        def start_i(t):
            b = t % NBUF
            return pltpu.async_copy(
                idx_hbm.at[pl.ds(base(t), CHUNK)], idx_v.at[b], isems[b])

        def start_g(t):
            b = t % NBUF
            return pltpu.async_copy(
                table_hbm.at[idx_v.at[b]], rows_v.at[b], gsems[b])

        def start_o(t):
            b = t % NBUF
            return pltpu.async_copy(
                rows_v.at[b], out_hbm.at[pl.ds(base(t), CHUNK)], osems[b])

        LAG = 2
        icopies = [start_i(t) for t in range(min(NBUF, trips))]
        gcopies = [None] * trips
        ocopies = [None] * trips
        for t in range(trips):
            if t >= NBUF:
                ocopies[t - NBUF].wait()
            icopies[t].wait()
            gcopies[t] = start_g(t)
            if t >= LAG:
                gcopies[t - LAG].wait()
                ocopies[t - LAG] = start_o(t - LAG)
                if t - LAG + NBUF < trips:
                    icopies.append(start_i(t - LAG + NBUF))
        for t in range(max(0, trips - LAG), trips):
            gcopies[t].wait()
            ocopies[t] = start_o(t)
        for t in range(max(0, trips - NBUF), trips):
            ocopies[t].wait()

    return gather_kernel


def kernel(x, emb_weight):
    act = _activate_table(emb_weight)
    idx = x.astype(jnp.int32)
    return _make_gather(idx.shape[0], emb_weight.shape[1])(act, idx)


# speedup vs baseline: 2.9381x; 1.9967x over previous
import functools

import jax
import jax.numpy as jnp
from jax import lax
from jax.experimental import pallas as pl
from jax.experimental.pallas import tpu as pltpu
from jax.experimental.pallas import tpu_sc as plsc


REPL = 16


def _swish_kernel(w_ref, o_ref):
    w = w_ref[...]
    o_ref[...] = jnp.broadcast_to((w * jax.nn.sigmoid(w))[None], o_ref.shape)


def _activate_table(w):
    r, c = w.shape
    out = pl.pallas_call(
        _swish_kernel,
        out_shape=jax.ShapeDtypeStruct((REPL, r, c), w.dtype),
    )(w)
    return out.reshape(REPL * r, c)


CHUNK = 128
NBUF = 4


@functools.lru_cache(maxsize=None)
def _make_gather(n, d, v):
    info = plsc.get_sparse_core_info()
    nc, ns = info.num_cores, info.num_subcores
    nw = nc * ns
    assert n % 8 == 0 and n >= CHUNK
    n_chunks = -(-n // CHUNK)
    trips = -(-n_chunks // nw)
    mesh = plsc.VectorSubcoreMesh(core_axis_name="c", subcore_axis_name="s")

    @functools.partial(
        pl.kernel,
        out_type=jax.ShapeDtypeStruct((n, d), jnp.float32),
        mesh=mesh,
        scratch_types=[
            pltpu.VMEM((NBUF, CHUNK), jnp.int32),
            pltpu.VMEM((NBUF, CHUNK, d), jnp.float32),
            [pltpu.SemaphoreType.DMA] * NBUF,
            [pltpu.SemaphoreType.DMA] * NBUF,
            [pltpu.SemaphoreType.DMA] * NBUF,
        ],
    )
    def gather_kernel(table_hbm, idx_hbm, out_hbm, idx_v, rows_v,
                      isems, gsems, osems):
        wid = lax.axis_index("s") * nc + lax.axis_index("c")
        offv = jnp.full((16,), (wid % REPL) * v, jnp.int32)

        def adjust(t):
            b = t % NBUF
            for c in range(0, CHUNK, 16):
                idx_v[b, pl.ds(c, 16)] = idx_v[b, pl.ds(c, 16)] + offv

        def base(t):
            j = wid + t * nw
            if (t + 1) * nw > n_chunks:
                j = jnp.where(j < n_chunks, j, wid)
            b = j * CHUNK
            if n % CHUNK != 0:
                b = jnp.minimum(b, n - CHUNK)
            return pl.multiple_of(b, 8)

        def start_i(t):
            b = t % NBUF
            return pltpu.async_copy(
                idx_hbm.at[pl.ds(base(t), CHUNK)], idx_v.at[b], isems[b])

        def start_g(t):
            b = t % NBUF
            return pltpu.async_copy(
                table_hbm.at[idx_v.at[b]], rows_v.at[b], gsems[b])

        def start_o(t):
            b = t % NBUF
            return pltpu.async_copy(
                rows_v.at[b], out_hbm.at[pl.ds(base(t), CHUNK)], osems[b])

        LAG = 2
        icopies = [start_i(t) for t in range(min(NBUF, trips))]
        gcopies = [None] * trips
        ocopies = [None] * trips
        for t in range(trips):
            if t >= NBUF:
                ocopies[t - NBUF].wait()
            icopies[t].wait()
            adjust(t)
            gcopies[t] = start_g(t)
            if t >= LAG:
                gcopies[t - LAG].wait()
                ocopies[t - LAG] = start_o(t - LAG)
                if t - LAG + NBUF < trips:
                    icopies.append(start_i(t - LAG + NBUF))
        for t in range(max(0, trips - LAG), trips):
            gcopies[t].wait()
            ocopies[t] = start_o(t)
        for t in range(max(0, trips - NBUF), trips):
            ocopies[t].wait()

    return gather_kernel


def kernel(x, emb_weight):
    act = _activate_table(emb_weight)
    idx = x.astype(jnp.int32)
    return _make_gather(idx.shape[0], emb_weight.shape[1],
                        emb_weight.shape[0])(act, idx)


# speedup vs baseline: 3.3002x; 1.1232x over previous
import functools

import jax
import jax.numpy as jnp
from jax import lax
from jax.experimental import pallas as pl
from jax.experimental.pallas import tpu as pltpu
from jax.experimental.pallas import tpu_sc as plsc


REPL = 32


def _swish_kernel(w_ref, o_ref):
    w = w_ref[...]
    o_ref[...] = jnp.broadcast_to((w * jax.nn.sigmoid(w))[None], o_ref.shape)


def _activate_table(w):
    r, c = w.shape
    out = pl.pallas_call(
        _swish_kernel,
        out_shape=jax.ShapeDtypeStruct((REPL, r, c), w.dtype),
    )(w)
    return out.reshape(REPL * r, c)


CHUNK = 128
NBUF = 4


@functools.lru_cache(maxsize=None)
def _make_gather(n, d, v):
    info = plsc.get_sparse_core_info()
    nc, ns = info.num_cores, info.num_subcores
    nw = nc * ns
    assert n % 8 == 0 and n >= CHUNK
    n_chunks = -(-n // CHUNK)
    trips = -(-n_chunks // nw)
    mesh = plsc.VectorSubcoreMesh(core_axis_name="c", subcore_axis_name="s")

    @functools.partial(
        pl.kernel,
        out_type=jax.ShapeDtypeStruct((n, d), jnp.float32),
        mesh=mesh,
        scratch_types=[
            pltpu.VMEM((NBUF, CHUNK), jnp.int32),
            pltpu.VMEM((NBUF, CHUNK, d), jnp.float32),
            [pltpu.SemaphoreType.DMA] * NBUF,
            [pltpu.SemaphoreType.DMA] * NBUF,
            [pltpu.SemaphoreType.DMA] * NBUF,
        ],
    )
    def gather_kernel(table_hbm, idx_hbm, out_hbm, idx_v, rows_v,
                      isems, gsems, osems):
        wid = lax.axis_index("s") * nc + lax.axis_index("c")
        offv = jnp.full((16,), (wid % REPL) * v, jnp.int32)

        def adjust(t):
            b = t % NBUF
            for c in range(0, CHUNK, 16):
                idx_v[b, pl.ds(c, 16)] = idx_v[b, pl.ds(c, 16)] + offv

        def base(t):
            j = wid + t * nw
            if (t + 1) * nw > n_chunks:
                j = jnp.where(j < n_chunks, j, wid)
            b = j * CHUNK
            if n % CHUNK != 0:
                b = jnp.minimum(b, n - CHUNK)
            return pl.multiple_of(b, 8)

        def start_i(t):
            b = t % NBUF
            return pltpu.async_copy(
                idx_hbm.at[pl.ds(base(t), CHUNK)], idx_v.at[b], isems[b])

        def start_g(t):
            b = t % NBUF
            return pltpu.async_copy(
                table_hbm.at[idx_v.at[b]], rows_v.at[b], gsems[b])

        def start_o(t):
            b = t % NBUF
            return pltpu.async_copy(
                rows_v.at[b], out_hbm.at[pl.ds(base(t), CHUNK)], osems[b])

        LAG = 2
        icopies = [start_i(t) for t in range(min(NBUF, trips))]
        gcopies = [None] * trips
        ocopies = [None] * trips
        for t in range(trips):
            if t >= NBUF:
                ocopies[t - NBUF].wait()
            icopies[t].wait()
            adjust(t)
            gcopies[t] = start_g(t)
            if t >= LAG:
                gcopies[t - LAG].wait()
                ocopies[t - LAG] = start_o(t - LAG)
                if t - LAG + NBUF < trips:
                    icopies.append(start_i(t - LAG + NBUF))
        for t in range(max(0, trips - LAG), trips):
            gcopies[t].wait()
            ocopies[t] = start_o(t)
        for t in range(max(0, trips - NBUF), trips):
            ocopies[t].wait()

    return gather_kernel


def kernel(x, emb_weight):
    act = _activate_table(emb_weight)
    idx = x.astype(jnp.int32)
    return _make_gather(idx.shape[0], emb_weight.shape[1],
                        emb_weight.shape[0])(act, idx)


# speedup vs baseline: 5.4478x; 1.6507x over previous
import functools

import jax
import jax.numpy as jnp
from jax import lax
from jax.experimental import pallas as pl
from jax.experimental.pallas import tpu as pltpu
from jax.experimental.pallas import tpu_sc as plsc


REPL = 32


def _swish_kernel(w_ref, o_ref):
    w = w_ref[...]
    o_ref[...] = jnp.broadcast_to((w * jax.nn.sigmoid(w))[None], o_ref.shape)


def _activate_table(w):
    r, c = w.shape
    out = pl.pallas_call(
        _swish_kernel,
        out_shape=jax.ShapeDtypeStruct((REPL, r, c), w.dtype),
    )(w)
    return out.reshape(REPL * r, c)


CHUNK = 128
NBUF = 4


@functools.lru_cache(maxsize=None)
def _make_gather(n, d, v):
    info = plsc.get_sparse_core_info()
    nc, ns = info.num_cores, info.num_subcores
    nw = nc * ns
    assert n % 8 == 0 and n >= CHUNK
    n_chunks = -(-n // CHUNK)
    trips = -(-n_chunks // nw)
    mesh = plsc.VectorSubcoreMesh(core_axis_name="c", subcore_axis_name="s")

    @functools.partial(
        pl.kernel,
        out_type=jax.ShapeDtypeStruct((n, d), jnp.float32),
        mesh=mesh,
        scratch_types=[
            pltpu.VMEM((NBUF, CHUNK), jnp.int32),
            pltpu.VMEM((NBUF, CHUNK, d), jnp.float32),
            pltpu.VMEM_SHARED((REPL * v, d), jnp.float32),
            [pltpu.SemaphoreType.DMA] * NBUF,
            [pltpu.SemaphoreType.DMA] * NBUF,
            [pltpu.SemaphoreType.DMA] * NBUF,
        ],
    )
    def gather_kernel(table_hbm, idx_hbm, out_hbm, idx_v, rows_v,
                      spm_tab, isems, gsems, osems):
        wid = lax.axis_index("s") * nc + lax.axis_index("c")
        offv = jnp.full((16,), (wid % REPL) * v, jnp.int32)

        @pl.when(lax.axis_index("s") == 0)
        def _():
            pltpu.sync_copy(table_hbm, spm_tab)

        plsc.subcore_barrier()

        def adjust(t):
            b = t % NBUF
            for c in range(0, CHUNK, 16):
                idx_v[b, pl.ds(c, 16)] = idx_v[b, pl.ds(c, 16)] + offv

        def base(t):
            j = wid + t * nw
            if (t + 1) * nw > n_chunks:
                j = jnp.where(j < n_chunks, j, wid)
            b = j * CHUNK
            if n % CHUNK != 0:
                b = jnp.minimum(b, n - CHUNK)
            return pl.multiple_of(b, 8)

        def start_i(t):
            b = t % NBUF
            return pltpu.async_copy(
                idx_hbm.at[pl.ds(base(t), CHUNK)], idx_v.at[b], isems[b])

        def start_g(t):
            b = t % NBUF
            return pltpu.async_copy(
                spm_tab.at[idx_v.at[b]], rows_v.at[b], gsems[b])

        def start_o(t):
            b = t % NBUF
            return pltpu.async_copy(
                rows_v.at[b], out_hbm.at[pl.ds(base(t), CHUNK)], osems[b])

        LAG = 2
        icopies = [start_i(t) for t in range(min(NBUF, trips))]
        gcopies = [None] * trips
        ocopies = [None] * trips
        for t in range(trips):
            if t >= NBUF:
                ocopies[t - NBUF].wait()
            icopies[t].wait()
            adjust(t)
            gcopies[t] = start_g(t)
            if t >= LAG:
                gcopies[t - LAG].wait()
                ocopies[t - LAG] = start_o(t - LAG)
                if t - LAG + NBUF < trips:
                    icopies.append(start_i(t - LAG + NBUF))
        for t in range(max(0, trips - LAG), trips):
            gcopies[t].wait()
            ocopies[t] = start_o(t)
        for t in range(max(0, trips - NBUF), trips):
            ocopies[t].wait()

    return gather_kernel


def kernel(x, emb_weight):
    act = _activate_table(emb_weight)
    idx = x.astype(jnp.int32)
    return _make_gather(idx.shape[0], emb_weight.shape[1],
                        emb_weight.shape[0])(act, idx)
